# baseline (speedup 1.0000x reference)
import functools

import jax
import jax.numpy as jnp
from jax import lax
from jax.experimental import pallas as pl
from jax.experimental.pallas import tpu as pltpu
from jax.experimental.pallas import tpu_sc as plsc

_B, _F, _V, _D = 16384, 26, 100001, 64
_NC, _NS = 2, 16
_NW = _NC * _NS
_TOTAL = _B * _F
_PER_W = _TOTAL // _NW
_CHUNK = 128
_NCHUNK = _PER_W // _CHUNK

_mesh = plsc.VectorSubcoreMesh(core_axis_name="c", subcore_axis_name="s")


@functools.partial(
    pl.kernel,
    mesh=_mesh,
    out_type=jax.ShapeDtypeStruct((_TOTAL, _D), jnp.float32),
    scratch_types=[
        pltpu.VMEM((1, _CHUNK), jnp.int32),
        pltpu.VMEM((1, _CHUNK, _D), jnp.float32),
        pltpu.SemaphoreType.DMA,
    ],
    compiler_params=pltpu.CompilerParams(use_tc_tiling_on_sc=False),
)
def _gather(x_hbm, table_hbm, out_hbm, idx_v, rows_v, sem):
    wid = lax.axis_index("s") * _NC + lax.axis_index("c")
    base = wid * _PER_W

    def chunk(c, carry):
        off = base + c * _CHUNK
        idx = idx_v.at[0]
        pltpu.sync_copy(x_hbm.at[pl.ds(off, _CHUNK)], idx)
        for i in range(_CHUNK // 16):
            lanes = lax.iota(jnp.int32, 16) + (off + i * 16)
            foff = lax.rem(lanes, _F) * _V
            sl = pl.ds(i * 16, 16)
            idx[sl] = idx[sl] + foff
        pltpu.async_copy(table_hbm.at[idx], rows_v.at[0], sem).wait()
        pltpu.sync_copy(rows_v.at[0], out_hbm.at[pl.ds(off, _CHUNK)])
        return carry

    lax.fori_loop(0, _NCHUNK, chunk, 0)


def kernel(x, tables):
    flat_x = x.reshape(_TOTAL)
    flat_tables = tables.reshape(_F * _V, _D)
    out = _gather(flat_x, flat_tables)
    return out.reshape(_B, _F, _D), (x == 0)


# speedup vs baseline: 1.0175x; 1.0175x over previous
import functools

import jax
import jax.numpy as jnp
from jax import lax
from jax.experimental import pallas as pl
from jax.experimental.pallas import tpu as pltpu
from jax.experimental.pallas import tpu_sc as plsc

_B, _F, _V, _D = 16384, 26, 100001, 64
_NC, _NS = 2, 16
_NW = _NC * _NS
_TOTAL = _B * _F
_PER_W = _TOTAL // _NW
_CHUNK = 128
_NCHUNK = _PER_W // _CHUNK
_NBUF = 8
_LAG = 4
_ROUNDS = _NCHUNK // _NBUF

_mesh = plsc.VectorSubcoreMesh(core_axis_name="c", subcore_axis_name="s")


@functools.partial(
    pl.kernel,
    mesh=_mesh,
    out_type=jax.ShapeDtypeStruct((_TOTAL, _D), jnp.float32),
    scratch_types=[
        pltpu.VMEM((_NCHUNK, _CHUNK), jnp.int32),
        pltpu.VMEM((_NBUF, _CHUNK, _D), jnp.float32),
        pltpu.SemaphoreType.DMA((_NBUF,)),
        pltpu.SemaphoreType.DMA((_NBUF,)),
    ],
    compiler_params=pltpu.CompilerParams(use_tc_tiling_on_sc=False),
)
def _gather(x_hbm, table_hbm, out_hbm, idx_v, rows_v, gsem, ssem):
    wid = lax.axis_index("s") * _NC + lax.axis_index("c")
    base = wid * _PER_W

    pltpu.sync_copy(x_hbm.at[pl.ds(wid * _NCHUNK, _NCHUNK)], idx_v)

    def idxmath(c, carry):
        row = idx_v.at[c]
        for i in range(_CHUNK // 16):
            lanes = lax.iota(jnp.int32, 16) + (c * _CHUNK + i * 16)
            sl = pl.ds(i * 16, 16)
            row[sl] = row[sl] + lax.rem(lanes, _F) * _V
        return carry

    lax.fori_loop(0, _NCHUNK, idxmath, 0)

    def fire_gather(c, b):
        pltpu.async_copy(table_hbm.at[idx_v.at[c]], rows_v.at[b], gsem.at[b])

    def wait_gather(c, b):
        pltpu.make_async_copy(
            table_hbm.at[idx_v.at[c]], rows_v.at[b], gsem.at[b]).wait()

    def fire_store(c, b):
        pltpu.async_copy(
            rows_v.at[b], out_hbm.at[pl.ds(base + c * _CHUNK, _CHUNK)],
            ssem.at[b])

    def wait_store(b):
        pltpu.make_async_copy(
            rows_v.at[b], out_hbm.at[pl.ds(base, _CHUNK)], ssem.at[b]).wait()

    for g in range(_LAG):
        fire_gather(g, g)

    def round_body(r, carry):
        for b in range(_NBUF):
            c = r * _NBUF + b
            g = c + _LAG
            sb = (b + _LAG) % _NBUF

            @pl.when(g < _NCHUNK)
            def _():
                @pl.when(g >= _NBUF)
                def _():
                    wait_store(sb)

                fire_gather(g, sb)

            wait_gather(c, b)
            fire_store(c, b)
        return carry

    lax.fori_loop(0, _ROUNDS, round_body, 0)

    for b in range(_NBUF):
        wait_store(b)


def kernel(x, tables):
    flat_x = x.reshape(_NW * _NCHUNK, _CHUNK)
    flat_tables = tables.reshape(_F * _V, _D)
    out = _gather(flat_x, flat_tables)
    return out.reshape(_B, _F, _D), (x == 0)
